# baseline (speedup 1.0000x reference)
import functools

import jax
import jax.numpy as jnp
from jax import lax
from jax.experimental import pallas as pl
from jax.experimental.pallas import tpu as pltpu
from jax.experimental.pallas import tpu_sc as plsc

N = 10000
E = 320000
D = 128

CB = 128
NCHUNK = E // CB
NWORKERS = 32
CPT = 81
NSLOT = 3
ROWS_PER_TILE = 624
TAIL_ROWS = N - 15 * ROWS_PER_TILE

_sc_mesh = plsc.VectorSubcoreMesh(core_axis_name="c", subcore_axis_name="s")


@functools.partial(
    pl.kernel,
    out_type=(
        jax.ShapeDtypeStruct((2, N, D), jnp.float32),
        jax.ShapeDtypeStruct((E, D), jnp.float32),
    ),
    mesh=_sc_mesh,
    scratch_types=[
        pltpu.VMEM((NSLOT, CB), jnp.int32),
        pltpu.VMEM((NSLOT, CB, D), jnp.float32),
        pltpu.VMEM_SHARED((N, D), jnp.float32),
        pltpu.SemaphoreType.DMA,
        pltpu.SemaphoreType.DMA,
        pltpu.SemaphoreType.DMA,
        pltpu.SemaphoreType.DMA,
        pltpu.SemaphoreType.DMA,
        pltpu.SemaphoreType.DMA,
    ],
)
def _sc_scatter(ef_hbm, dst_hbm, out_hbm, ecopy_hbm, idx_v, ebuf, acc,
                dsem0, dsem1, dsem2, wsem0, wsem1, wsem2):
    cid = lax.axis_index("c")
    sid = lax.axis_index("s")
    w = cid * 16 + sid
    dsems = (dsem0, dsem1, dsem2)
    wsems = (wsem0, wsem1, wsem2)

    zvec = jnp.zeros((16,), jnp.float32)

    def zrow(r, carry):
        for d8 in range(D // 16):
            ebuf[0, r, pl.ds(d8 * 16, 16)] = zvec
        return carry

    lax.fori_loop(0, CB, zrow, 0)

    for q in range(4):
        pltpu.sync_copy(
            ebuf.at[0], acc.at[pl.ds(sid * ROWS_PER_TILE + q * CB, CB)]
        )

    @pl.when(sid < 15)
    def _():
        pltpu.sync_copy(
            ebuf.at[0, pl.ds(0, ROWS_PER_TILE - 4 * CB)],
            acc.at[pl.ds(sid * ROWS_PER_TILE + 4 * CB, ROWS_PER_TILE - 4 * CB)],
        )

    @pl.when(sid == 15)
    def _():
        pltpu.sync_copy(
            ebuf.at[0], acc.at[pl.ds(15 * ROWS_PER_TILE + 4 * CB, CB)]
        )

    plsc.subcore_barrier()

    def chunk_of(k):
        return w + NWORKERS * k

    def start_dma(k, slot):
        c = chunk_of(k)

        @pl.when(c < NCHUNK)
        def _():
            pltpu.make_async_copy(
                dst_hbm.at[pl.ds(c * CB, CB)], idx_v.at[slot], dsems[slot]
            ).start()
            pltpu.make_async_copy(
                ef_hbm.at[pl.ds(c * CB, CB)], ebuf.at[slot], dsems[slot]
            ).start()

    def wait_dma(k, slot):
        c = chunk_of(k)

        @pl.when(c < NCHUNK)
        def _():
            pltpu.make_async_copy(
                dst_hbm.at[pl.ds(c * CB, CB)], idx_v.at[slot], dsems[slot]
            ).wait()
            pltpu.make_async_copy(
                ef_hbm.at[pl.ds(c * CB, CB)], ebuf.at[slot], dsems[slot]
            ).wait()

    def scatter(k, slot):
        c = chunk_of(k)

        @pl.when(c < NCHUNK)
        def _():
            pltpu.sync_copy(ebuf.at[slot], acc.at[idx_v.at[slot]], add=True)

    def start_wb(k, slot):
        c = chunk_of(k)

        @pl.when(c < NCHUNK)
        def _():
            pltpu.make_async_copy(
                ebuf.at[slot], ecopy_hbm.at[pl.ds(c * CB, CB)], wsems[slot]
            ).start()

    def wait_wb(k, slot):
        c = chunk_of(k)

        @pl.when((k >= 0) & (c < NCHUNK))
        def _():
            pltpu.make_async_copy(
                ebuf.at[slot], ecopy_hbm.at[pl.ds(c * CB, CB)], wsems[slot]
            ).wait()

    start_dma(0, 0)
    start_dma(1, 1)

    def group_body(i, carry):
        for u in range(NSLOT):
            k = NSLOT * i + u
            wait_dma(k, u)
            start_wb(k, u)
            scatter(k, u)
            wait_wb(k - 1, (u + 2) % NSLOT)
            start_dma(k + 2, (u + 2) % NSLOT)
        return carry

    lax.fori_loop(0, CPT // NSLOT, group_body, 0)

    wait_wb(CPT - 1, (CPT - 1) % NSLOT)

    plsc.subcore_barrier()

    @pl.when(sid < 15)
    def _():
        pltpu.sync_copy(
            acc.at[pl.ds(sid * ROWS_PER_TILE, ROWS_PER_TILE)],
            out_hbm.at[cid, pl.ds(sid * ROWS_PER_TILE, ROWS_PER_TILE)],
        )

    @pl.when(sid == 15)
    def _():
        pltpu.sync_copy(
            acc.at[pl.ds(15 * ROWS_PER_TILE, TAIL_ROWS)],
            out_hbm.at[cid, pl.ds(15 * ROWS_PER_TILE, TAIL_ROWS)],
        )


BR = 2000


def _mlp_body(parts_ref, nfeat_ref, w1_ref, b1_ref, w2_ref, b2_ref, g_ref,
              bt_ref, out_ref):
    agg = parts_ref[0] + parts_ref[1]
    x = nfeat_ref[...]
    h = (
        jnp.dot(agg, w1_ref[:D, :], preferred_element_type=jnp.float32)
        + jnp.dot(x, w1_ref[D:, :], preferred_element_type=jnp.float32)
        + b1_ref[...]
    )
    h = h * jax.nn.sigmoid(h)
    h = jnp.dot(h, w2_ref[...], preferred_element_type=jnp.float32) + b2_ref[...]
    mean = jnp.mean(h, axis=-1, keepdims=True)
    var = jnp.mean((h - mean) * (h - mean), axis=-1, keepdims=True)
    h = (h - mean) * jax.lax.rsqrt(var + 1e-5) * g_ref[...] + bt_ref[...]
    out_ref[...] = h + x


def _mlp(parts, nfeat, W1, b1, W2, b2, gamma, beta):
    grid = N // BR
    return pl.pallas_call(
        _mlp_body,
        grid=(grid,),
        in_specs=[
            pl.BlockSpec((2, BR, D), lambda i: (0, i, 0)),
            pl.BlockSpec((BR, D), lambda i: (i, 0)),
            pl.BlockSpec((2 * D, D), lambda i: (0, 0)),
            pl.BlockSpec((1, D), lambda i: (0, 0)),
            pl.BlockSpec((D, D), lambda i: (0, 0)),
            pl.BlockSpec((1, D), lambda i: (0, 0)),
            pl.BlockSpec((1, D), lambda i: (0, 0)),
            pl.BlockSpec((1, D), lambda i: (0, 0)),
        ],
        out_specs=pl.BlockSpec((BR, D), lambda i: (i, 0)),
        out_shape=jax.ShapeDtypeStruct((N, D), jnp.float32),
    )(parts, nfeat, W1, b1.reshape(1, D), W2, b2.reshape(1, D),
      gamma.reshape(1, D), beta.reshape(1, D))


def kernel(efeat, nfeat, dst_idx, W1, b1, W2, b2, gamma, beta):
    parts, efeat_out = _sc_scatter(efeat, dst_idx)
    nfeat_new = _mlp(parts, nfeat, W1, b1, W2, b2, gamma, beta)
    return (efeat_out, nfeat_new)


# speedup vs baseline: 1.0109x; 1.0109x over previous
import functools

import jax
import jax.numpy as jnp
from jax import lax
from jax.experimental import pallas as pl
from jax.experimental.pallas import tpu as pltpu
from jax.experimental.pallas import tpu_sc as plsc

N = 10000
E = 320000
D = 128

CB = 128
NCHUNK = E // CB
NWORKERS = 32
CPT = 81
NSLOT = 3
ROWS_PER_TILE = 624
TAIL_ROWS = N - 15 * ROWS_PER_TILE

_sc_mesh = plsc.VectorSubcoreMesh(core_axis_name="c", subcore_axis_name="s")


@functools.partial(
    pl.kernel,
    out_type=(
        jax.ShapeDtypeStruct((2, N, D), jnp.float32),
        jax.ShapeDtypeStruct((E, D), jnp.float32),
    ),
    mesh=_sc_mesh,
    scratch_types=[
        pltpu.VMEM((NSLOT, CB), jnp.int32),
        pltpu.VMEM((NSLOT, CB, D), jnp.float32),
        pltpu.VMEM_SHARED((N, D), jnp.float32),
        pltpu.SemaphoreType.DMA,
        pltpu.SemaphoreType.DMA,
        pltpu.SemaphoreType.DMA,
        pltpu.SemaphoreType.DMA,
        pltpu.SemaphoreType.DMA,
        pltpu.SemaphoreType.DMA,
    ],
)
def _sc_scatter(ef_hbm, dst_hbm, out_hbm, ecopy_hbm, idx_v, ebuf, acc,
                dsem0, dsem1, dsem2, wsem0, wsem1, wsem2):
    cid = lax.axis_index("c")
    sid = lax.axis_index("s")
    w = cid * 16 + sid
    dsems = (dsem0, dsem1, dsem2)
    wsems = (wsem0, wsem1, wsem2)

    def chunk_of(k):
        return w + NWORKERS * k

    def start_dma(k, slot):
        c = chunk_of(k)

        @pl.when(c < NCHUNK)
        def _():
            pltpu.make_async_copy(
                dst_hbm.at[pl.ds(c * CB, CB)], idx_v.at[slot], dsems[slot]
            ).start()
            pltpu.make_async_copy(
                ef_hbm.at[pl.ds(c * CB, CB)], ebuf.at[slot], dsems[slot]
            ).start()

    def wait_dma(k, slot):
        c = chunk_of(k)

        @pl.when(c < NCHUNK)
        def _():
            pltpu.make_async_copy(
                dst_hbm.at[pl.ds(c * CB, CB)], idx_v.at[slot], dsems[slot]
            ).wait()
            pltpu.make_async_copy(
                ef_hbm.at[pl.ds(c * CB, CB)], ebuf.at[slot], dsems[slot]
            ).wait()

    def scatter(k, slot):
        c = chunk_of(k)

        @pl.when(c < NCHUNK)
        def _():
            pltpu.sync_copy(ebuf.at[slot], acc.at[idx_v.at[slot]], add=True)

    def start_wb(k, slot):
        c = chunk_of(k)

        @pl.when(c < NCHUNK)
        def _():
            pltpu.make_async_copy(
                ebuf.at[slot], ecopy_hbm.at[pl.ds(c * CB, CB)], wsems[slot]
            ).start()

    def wait_wb(k, slot):
        c = chunk_of(k)

        @pl.when((k >= 0) & (c < NCHUNK))
        def _():
            pltpu.make_async_copy(
                ebuf.at[slot], ecopy_hbm.at[pl.ds(c * CB, CB)], wsems[slot]
            ).wait()

    start_dma(0, 0)
    start_dma(1, 1)

    zvec = jnp.zeros((16,), jnp.float32)

    def zrow(r, carry):
        for d8 in range(D // 16):
            ebuf[2, r, pl.ds(d8 * 16, 16)] = zvec
        return carry

    lax.fori_loop(0, CB, zrow, 0)

    for q in range(4):
        pltpu.sync_copy(
            ebuf.at[2], acc.at[pl.ds(sid * ROWS_PER_TILE + q * CB, CB)]
        )

    @pl.when(sid < 15)
    def _():
        pltpu.sync_copy(
            ebuf.at[2, pl.ds(0, ROWS_PER_TILE - 4 * CB)],
            acc.at[pl.ds(sid * ROWS_PER_TILE + 4 * CB, ROWS_PER_TILE - 4 * CB)],
        )

    @pl.when(sid == 15)
    def _():
        pltpu.sync_copy(
            ebuf.at[2], acc.at[pl.ds(15 * ROWS_PER_TILE + 4 * CB, CB)]
        )

    plsc.subcore_barrier()

    def group_body(i, carry):
        for u in range(NSLOT):
            k = NSLOT * i + u
            wait_dma(k, u)
            start_wb(k, u)
            scatter(k, u)
            wait_wb(k - 1, (u + 2) % NSLOT)
            start_dma(k + 2, (u + 2) % NSLOT)
        return carry

    lax.fori_loop(0, CPT // NSLOT, group_body, 0)

    wait_wb(CPT - 1, (CPT - 1) % NSLOT)

    plsc.subcore_barrier()

    @pl.when(sid < 15)
    def _():
        pltpu.sync_copy(
            acc.at[pl.ds(sid * ROWS_PER_TILE, ROWS_PER_TILE)],
            out_hbm.at[cid, pl.ds(sid * ROWS_PER_TILE, ROWS_PER_TILE)],
        )

    @pl.when(sid == 15)
    def _():
        pltpu.sync_copy(
            acc.at[pl.ds(15 * ROWS_PER_TILE, TAIL_ROWS)],
            out_hbm.at[cid, pl.ds(15 * ROWS_PER_TILE, TAIL_ROWS)],
        )


BR = 2000


def _mlp_body(parts_ref, nfeat_ref, w1_ref, b1_ref, w2_ref, b2_ref, g_ref,
              bt_ref, out_ref):
    agg = parts_ref[0] + parts_ref[1]
    x = nfeat_ref[...]
    h = (
        jnp.dot(agg, w1_ref[:D, :], preferred_element_type=jnp.float32)
        + jnp.dot(x, w1_ref[D:, :], preferred_element_type=jnp.float32)
        + b1_ref[...]
    )
    h = h * jax.nn.sigmoid(h)
    h = jnp.dot(h, w2_ref[...], preferred_element_type=jnp.float32) + b2_ref[...]
    mean = jnp.mean(h, axis=-1, keepdims=True)
    var = jnp.mean((h - mean) * (h - mean), axis=-1, keepdims=True)
    h = (h - mean) * jax.lax.rsqrt(var + 1e-5) * g_ref[...] + bt_ref[...]
    out_ref[...] = h + x


def _mlp(parts, nfeat, W1, b1, W2, b2, gamma, beta):
    grid = N // BR
    return pl.pallas_call(
        _mlp_body,
        grid=(grid,),
        in_specs=[
            pl.BlockSpec((2, BR, D), lambda i: (0, i, 0)),
            pl.BlockSpec((BR, D), lambda i: (i, 0)),
            pl.BlockSpec((2 * D, D), lambda i: (0, 0)),
            pl.BlockSpec((1, D), lambda i: (0, 0)),
            pl.BlockSpec((D, D), lambda i: (0, 0)),
            pl.BlockSpec((1, D), lambda i: (0, 0)),
            pl.BlockSpec((1, D), lambda i: (0, 0)),
            pl.BlockSpec((1, D), lambda i: (0, 0)),
        ],
        out_specs=pl.BlockSpec((BR, D), lambda i: (i, 0)),
        out_shape=jax.ShapeDtypeStruct((N, D), jnp.float32),
    )(parts, nfeat, W1, b1.reshape(1, D), W2, b2.reshape(1, D),
      gamma.reshape(1, D), beta.reshape(1, D))


def kernel(efeat, nfeat, dst_idx, W1, b1, W2, b2, gamma, beta):
    parts, efeat_out = _sc_scatter(efeat, dst_idx)
    nfeat_new = _mlp(parts, nfeat, W1, b1, W2, b2, gamma, beta)
    return (efeat_out, nfeat_new)


# speedup vs baseline: 1.0324x; 1.0212x over previous
import functools

import jax
import jax.numpy as jnp
from jax import lax
from jax.experimental import pallas as pl
from jax.experimental.pallas import tpu as pltpu
from jax.experimental.pallas import tpu_sc as plsc

N = 10000
E = 320000
D = 128

CB = 128
NCHUNK = E // CB
NWORKERS = 32
CPT = 81
NSLOT = 3
ROWS_PER_TILE = 624
TAIL_ROWS = N - 15 * ROWS_PER_TILE

_sc_mesh = plsc.VectorSubcoreMesh(core_axis_name="c", subcore_axis_name="s")


@functools.partial(
    pl.kernel,
    out_type=(
        jax.ShapeDtypeStruct((2, N, D), jnp.float32),
        jax.ShapeDtypeStruct((E, D), jnp.float32),
    ),
    mesh=_sc_mesh,
    scratch_types=[
        pltpu.VMEM((NSLOT, CB), jnp.int32),
        pltpu.VMEM((NSLOT, CB, D), jnp.float32),
        pltpu.VMEM_SHARED((N, D), jnp.float32),
        pltpu.SemaphoreType.DMA,
        pltpu.SemaphoreType.DMA,
        pltpu.SemaphoreType.DMA,
        pltpu.SemaphoreType.DMA,
        pltpu.SemaphoreType.DMA,
        pltpu.SemaphoreType.DMA,
    ],
)
def _sc_scatter(ef_hbm, dst_hbm, out_hbm, ecopy_hbm, idx_v, ebuf, acc,
                dsem0, dsem1, dsem2, wsem0, wsem1, wsem2):
    cid = lax.axis_index("c")
    sid = lax.axis_index("s")
    w = cid * 16 + sid
    dsems = (dsem0, dsem1, dsem2)
    wsems = (wsem0, wsem1, wsem2)

    def chunk_of(k):
        return w + NWORKERS * k

    def start_dma(k, slot):
        c = chunk_of(k)

        @pl.when(c < NCHUNK)
        def _():
            pltpu.make_async_copy(
                dst_hbm.at[pl.ds(c * CB, CB)], idx_v.at[slot], dsems[slot]
            ).start()
            pltpu.make_async_copy(
                ef_hbm.at[pl.ds(c * CB, CB)], ebuf.at[slot], dsems[slot]
            ).start()

    def wait_dma(k, slot):
        c = chunk_of(k)

        @pl.when(c < NCHUNK)
        def _():
            pltpu.make_async_copy(
                dst_hbm.at[pl.ds(c * CB, CB)], idx_v.at[slot], dsems[slot]
            ).wait()
            pltpu.make_async_copy(
                ef_hbm.at[pl.ds(c * CB, CB)], ebuf.at[slot], dsems[slot]
            ).wait()

    def scatter(k, slot):
        c = chunk_of(k)

        @pl.when(c < NCHUNK)
        def _():
            pltpu.sync_copy(ebuf.at[slot], acc.at[idx_v.at[slot]], add=True)

    def start_wb(k, slot):
        c = chunk_of(k)

        @pl.when(c < NCHUNK)
        def _():
            pltpu.make_async_copy(
                ebuf.at[slot], ecopy_hbm.at[pl.ds(c * CB, CB)], wsems[slot]
            ).start()

    def wait_wb(k, slot):
        c = chunk_of(k)

        @pl.when((k >= 0) & (c < NCHUNK))
        def _():
            pltpu.make_async_copy(
                ebuf.at[slot], ecopy_hbm.at[pl.ds(c * CB, CB)], wsems[slot]
            ).wait()

    start_dma(0, 0)
    start_dma(1, 1)

    zvec = jnp.zeros((16,), jnp.float32)

    def zrow(r, carry):
        for d8 in range(D // 16):
            ebuf[2, r, pl.ds(d8 * 16, 16)] = zvec
        return carry

    lax.fori_loop(0, CB, zrow, 0)

    for q in range(4):
        pltpu.sync_copy(
            ebuf.at[2], acc.at[pl.ds(sid * ROWS_PER_TILE + q * CB, CB)]
        )

    @pl.when(sid < 15)
    def _():
        pltpu.sync_copy(
            ebuf.at[2, pl.ds(0, ROWS_PER_TILE - 4 * CB)],
            acc.at[pl.ds(sid * ROWS_PER_TILE + 4 * CB, ROWS_PER_TILE - 4 * CB)],
        )

    @pl.when(sid == 15)
    def _():
        pltpu.sync_copy(
            ebuf.at[2], acc.at[pl.ds(15 * ROWS_PER_TILE + 4 * CB, CB)]
        )

    plsc.subcore_barrier()

    def group_body(i, carry):
        for u in range(NSLOT):
            k = NSLOT * i + u
            wait_dma(k, u)
            scatter(k, u)
            start_wb(k, u)
            wait_wb(k - 1, (u + 2) % NSLOT)
            start_dma(k + 2, (u + 2) % NSLOT)
        return carry

    lax.fori_loop(0, CPT // NSLOT, group_body, 0)

    wait_wb(CPT - 1, (CPT - 1) % NSLOT)

    plsc.subcore_barrier()

    @pl.when(sid < 15)
    def _():
        pltpu.sync_copy(
            acc.at[pl.ds(sid * ROWS_PER_TILE, ROWS_PER_TILE)],
            out_hbm.at[cid, pl.ds(sid * ROWS_PER_TILE, ROWS_PER_TILE)],
        )

    @pl.when(sid == 15)
    def _():
        pltpu.sync_copy(
            acc.at[pl.ds(15 * ROWS_PER_TILE, TAIL_ROWS)],
            out_hbm.at[cid, pl.ds(15 * ROWS_PER_TILE, TAIL_ROWS)],
        )


BR = 2000


def _mlp_body(parts_ref, nfeat_ref, w1_ref, b1_ref, w2_ref, b2_ref, g_ref,
              bt_ref, out_ref):
    agg = parts_ref[0] + parts_ref[1]
    x = nfeat_ref[...]
    h = (
        jnp.dot(agg, w1_ref[:D, :], preferred_element_type=jnp.float32)
        + jnp.dot(x, w1_ref[D:, :], preferred_element_type=jnp.float32)
        + b1_ref[...]
    )
    h = h * jax.nn.sigmoid(h)
    h = jnp.dot(h, w2_ref[...], preferred_element_type=jnp.float32) + b2_ref[...]
    mean = jnp.mean(h, axis=-1, keepdims=True)
    var = jnp.mean((h - mean) * (h - mean), axis=-1, keepdims=True)
    h = (h - mean) * jax.lax.rsqrt(var + 1e-5) * g_ref[...] + bt_ref[...]
    out_ref[...] = h + x


def _mlp(parts, nfeat, W1, b1, W2, b2, gamma, beta):
    grid = N // BR
    return pl.pallas_call(
        _mlp_body,
        grid=(grid,),
        in_specs=[
            pl.BlockSpec((2, BR, D), lambda i: (0, i, 0)),
            pl.BlockSpec((BR, D), lambda i: (i, 0)),
            pl.BlockSpec((2 * D, D), lambda i: (0, 0)),
            pl.BlockSpec((1, D), lambda i: (0, 0)),
            pl.BlockSpec((D, D), lambda i: (0, 0)),
            pl.BlockSpec((1, D), lambda i: (0, 0)),
            pl.BlockSpec((1, D), lambda i: (0, 0)),
            pl.BlockSpec((1, D), lambda i: (0, 0)),
        ],
        out_specs=pl.BlockSpec((BR, D), lambda i: (i, 0)),
        out_shape=jax.ShapeDtypeStruct((N, D), jnp.float32),
    )(parts, nfeat, W1, b1.reshape(1, D), W2, b2.reshape(1, D),
      gamma.reshape(1, D), beta.reshape(1, D))


def kernel(efeat, nfeat, dst_idx, W1, b1, W2, b2, gamma, beta):
    parts, efeat_out = _sc_scatter(efeat, dst_idx)
    nfeat_new = _mlp(parts, nfeat, W1, b1, W2, b2, gamma, beta)
    return (efeat_out, nfeat_new)
